# baseline (speedup 1.0000x reference)
import jax
import jax.numpy as jnp
from jax.experimental import pallas as pl

EPS = 1e-10

_ROWS = 128
_COLS = 100000
_BLOCK_ROWS = 8


def _gumbel_softmax_kernel(logits_ref, uniform_ref, out_ref):
    z = logits_ref[...] - jnp.log(EPS - jnp.log(uniform_ref[...] + EPS))
    m = jnp.max(z, axis=-1, keepdims=True)
    e = jnp.exp(z - m)
    s = jnp.sum(e, axis=-1, keepdims=True)
    out_ref[...] = e / s


def kernel(logits, uniform):
    grid = (_ROWS // _BLOCK_ROWS,)
    spec = pl.BlockSpec((_BLOCK_ROWS, _COLS), lambda i: (i, 0))
    return pl.pallas_call(
        _gumbel_softmax_kernel,
        grid=grid,
        in_specs=[spec, spec],
        out_specs=spec,
        out_shape=jax.ShapeDtypeStruct((_ROWS, _COLS), jnp.float32),
    )(logits, uniform)
